# baseline (speedup 1.0000x reference)
import functools

import jax
import jax.numpy as jnp
from jax import lax
from jax.experimental import pallas as pl
from jax.experimental.pallas import tpu as pltpu
from jax.experimental.pallas import tpu_sc as plsc

_NUM_CORES = 2
_NUM_SUBCORES = 16
_NUM_WORKERS = _NUM_CORES * _NUM_SUBCORES

_CHUNK = 128
_NBUF = 5


def _make_sc_gather(n_idx: int, vocab: int, emb: int):
    assert n_idx % (_NUM_WORKERS * _CHUNK * _NBUF) == 0
    per_w = n_idx // _NUM_WORKERS
    groups = per_w // (_CHUNK * _NBUF)

    mesh = plsc.VectorSubcoreMesh(core_axis_name="c", subcore_axis_name="s")

    @functools.partial(
        pl.kernel,
        mesh=mesh,
        out_type=jax.ShapeDtypeStruct((n_idx, emb), jnp.float32),
        compiler_params=pltpu.CompilerParams(use_tc_tiling_on_sc=False),
        scratch_types=[
            pltpu.VMEM((per_w,), jnp.int32),
            [pltpu.VMEM((_CHUNK, emb), jnp.float32) for _ in range(_NBUF)],
            [pltpu.SemaphoreType.DMA for _ in range(_NBUF)],
        ],
    )
    def gather_kernel(table_hbm, idx_hbm, out_hbm, idx_v, rows, sems):
        wid = lax.axis_index("s") * _NUM_CORES + lax.axis_index("c")
        base = wid * per_w
        pltpu.sync_copy(idx_hbm.at[pl.ds(base, per_w)], idx_v)

        def group_body(g, carry):
            goff = g * (_CHUNK * _NBUF)
            copies = []
            for b in range(_NBUF):
                off = goff + b * _CHUNK
                copies.append(
                    pltpu.async_copy(
                        table_hbm.at[idx_v.at[pl.ds(off, _CHUNK)]],
                        rows[b],
                        sems[b],
                    )
                )
            for b in range(_NBUF):
                off = goff + b * _CHUNK
                copies[b].wait()
                pltpu.sync_copy(rows[b], out_hbm.at[pl.ds(base + off, _CHUNK)])
            return carry

        lax.fori_loop(0, groups, group_body, 0)

    return gather_kernel


def _rnn_step(emb_ref, wih_ref, whh_ref, b_ref, out_ref, h_ref):
    t = pl.program_id(0)

    @pl.when(t == 0)
    def _():
        h_ref[...] = jnp.zeros_like(h_ref)

    x = emb_ref[0]
    h = h_ref[...]
    pre = (
        jnp.dot(x, wih_ref[...], preferred_element_type=jnp.float32)
        + jnp.dot(h, whh_ref[...], preferred_element_type=jnp.float32)
        + b_ref[...]
    )
    h_new = jnp.tanh(pre)
    h_ref[...] = h_new
    out_ref[0] = h_new


def _make_tc_rnn(seq: int, batch: int, emb: int, hid: int):
    return pl.pallas_call(
        _rnn_step,
        grid=(seq,),
        in_specs=[
            pl.BlockSpec((1, batch, emb), lambda t: (t, 0, 0)),
            pl.BlockSpec((emb, hid), lambda t: (0, 0)),
            pl.BlockSpec((hid, hid), lambda t: (0, 0)),
            pl.BlockSpec((1, hid), lambda t: (0, 0)),
        ],
        out_specs=pl.BlockSpec((1, batch, hid), lambda t: (t, 0, 0)),
        out_shape=jax.ShapeDtypeStruct((seq, batch, hid), jnp.float32),
        scratch_shapes=[pltpu.VMEM((batch, hid), jnp.float32)],
    )


def kernel(input_seq, emb_table, W_ih, W_hh, b_ih, b_hh):
    seq, batch = input_seq.shape
    vocab, emb = emb_table.shape
    hid = W_hh.shape[0]

    idx_flat = input_seq.reshape(-1).astype(jnp.int32)
    gathered = _make_sc_gather(seq * batch, vocab, emb)(emb_table, idx_flat)
    embedded_seq = gathered.reshape(seq, batch, emb)

    bias = (b_ih + b_hh).reshape(1, hid)
    output_seq = _make_tc_rnn(seq, batch, emb, hid)(
        embedded_seq, W_ih.T, W_hh.T, bias
    )
    last_hidden = output_seq[seq - 1 : seq]
    return output_seq, last_hidden, embedded_seq
